# baseline (speedup 1.0000x reference)
import functools

import jax
import jax.numpy as jnp
from jax import lax
from jax.experimental import pallas as pl
from jax.experimental.pallas import tpu as pltpu
from jax.experimental.pallas import tpu_sc as plsc

N = 10000
E = 320000
D = 128
DW = 16
DG = D + DW
GW = 80
NC = 2
NS = 16
NW = NC * NS
EPW = E // NW
KB = 80
NBLK = EPW // KB
ROWB = 400
GRID = N // ROWB
NP = 10240
ZROWS = 8


def _prep_body(x_ref, w0_ref, b0_ref, att_ref, g_ref):
    x = x_ref[...]
    fs = lax.dot_general(x, w0_ref[...], (((1,), (1,)), ((), ())),
                         preferred_element_type=jnp.float32) + b0_ref[...]
    el = jnp.sum(fs * att_ref[...], axis=1, keepdims=True)
    el = jnp.where(el >= 0.0, el, 0.2 * el)
    s = jnp.exp(el)
    g_ref[:, :D] = fs * s
    g_ref[:, D:DG] = jnp.broadcast_to(s, (ROWB, DW))


def _final_body(x_ref, p_ref, w1_ref, b1_ref, sc_ref, off_ref, o_ref):
    x = x_ref[...]
    fself = lax.dot_general(x, w1_ref[...], (((1,), (1,)), ((), ())),
                            preferred_element_type=jnp.float32) + b1_ref[...]
    p = p_ref[...]
    tot = p[0] + p[1]
    den = jnp.max(tot[:, D:DG], axis=1, keepdims=True)
    aggr = tot[:, :D] / jnp.maximum(den, 1e-10)
    feat = jnp.maximum(aggr + fself, 0.0)
    m = jnp.mean(feat, axis=1, keepdims=True)
    d = feat - m
    v = jnp.mean(d * d, axis=1, keepdims=True)
    o_ref[...] = d * lax.rsqrt(v + 1e-9) * sc_ref[...] + off_ref[...]


_sc_mesh = plsc.VectorSubcoreMesh(core_axis_name="c", subcore_axis_name="s")


BSLOT = 2
ISLOT = 4


@functools.partial(
    pl.kernel,
    mesh=_sc_mesh,
    out_type=jax.ShapeDtypeStruct((NC, NP, DG), jnp.float32),
    scratch_types=[
        pltpu.VMEM((ISLOT, KB), jnp.int32),
        pltpu.VMEM((ISLOT, KB), jnp.int32),
        pltpu.VMEM((ISLOT, KB), jnp.float32),
        pltpu.VMEM((KB, GW), jnp.int32),
        pltpu.VMEM((KB, GW), jnp.int32),
        pltpu.VMEM((KB, DG), jnp.float32),
        pltpu.VMEM((KB, DG), jnp.float32),
        pltpu.VMEM((ZROWS, DG), jnp.float32),
        pltpu.VMEM_SHARED((NP, DG), jnp.float32),
        pltpu.SemaphoreType.DMA,
        pltpu.SemaphoreType.DMA,
        pltpu.SemaphoreType.DMA,
        pltpu.SemaphoreType.DMA,
        pltpu.SemaphoreType.DMA,
        pltpu.SemaphoreType.DMA,
        pltpu.SemaphoreType.DMA,
        pltpu.SemaphoreType.DMA,
    ],
    compiler_params=pltpu.CompilerParams(use_tc_tiling_on_sc=False,
                                         needs_layout_passes=False),
)
def _agg(rows_hbm, cols_hbm, adj_hbm, g_hbm, out_hbm,
         rowb, colb, adjb, gbuf0, gbuf1, sbuf0, sbuf1, zbuf, ps,
         isem0, isem1, isem2, isem3, gsem0, gsem1, ssem0, ssem1):
    cid = lax.axis_index("c")
    sid = lax.axis_index("s")
    wid = sid * NC + cid
    gbufs = (gbuf0, gbuf1)
    sbufs = (sbuf0, sbuf1)
    isems = (isem0, isem1, isem2, isem3)
    gsems = (gsem0, gsem1)
    ssems = (ssem0, ssem1)
    zero16 = jnp.zeros((16,), jnp.float32)

    def zrow(i, carry):
        for q in range(DG // 16):
            zbuf[i, pl.ds(q * 16, 16)] = zero16
        return carry
    lax.fori_loop(0, ZROWS, zrow, 0)
    nrows = NP // NS

    def zchunk(k, carry):
        pltpu.sync_copy(zbuf, ps.at[pl.ds(sid * nrows + k * ZROWS, ZROWS)])
        return carry
    lax.fori_loop(0, nrows // ZROWS, zchunk, 0)
    plsc.subcore_barrier()

    def _eoff(j):
        return pl.multiple_of(wid * EPW + j * KB, 8)

    def idx_fetch(j, s):
        off = _eoff(j)
        pltpu.async_copy(rows_hbm.at[pl.ds(off, KB)], rowb.at[s], isems[s])
        pltpu.async_copy(cols_hbm.at[pl.ds(off, KB)], colb.at[s], isems[s])
        pltpu.async_copy(adj_hbm.at[pl.ds(off, KB)], adjb.at[s], isems[s])

    def idx_wait(j, s):
        off = _eoff(j)
        pltpu.make_async_copy(rows_hbm.at[pl.ds(off, KB)], rowb.at[s],
                              isems[s]).wait()
        pltpu.make_async_copy(cols_hbm.at[pl.ds(off, KB)], colb.at[s],
                              isems[s]).wait()
        pltpu.make_async_copy(adj_hbm.at[pl.ds(off, KB)], adjb.at[s],
                              isems[s]).wait()

    def gather_start(b2, i4):
        pltpu.async_copy(g_hbm.at[colb.at[i4]], gbufs[b2], gsems[b2])

    def gather_wait(b2, i4):
        pltpu.make_async_copy(g_hbm.at[colb.at[i4]], gbufs[b2],
                              gsems[b2]).wait()

    def scat_start(b2, i4):
        pltpu.async_copy(sbufs[b2], ps.at[rowb.at[i4]], ssems[b2], add=True)

    def scat_wait(b2, i4):
        pltpu.make_async_copy(sbufs[b2], ps.at[rowb.at[i4]], ssems[b2]).wait()

    HMASK = jnp.int32(-65536)

    def scale(b2, i4):
        gb = gbufs[b2]
        sb = sbufs[b2]

        def tbody(t, c):
            off = pl.multiple_of(t * 16, 16)
            av = adjb[i4, pl.ds(off, 16)]
            for i in range(16):
                wv = jnp.full((16,), av[i])
                r = t * 16 + i
                for q in range(GW // 16):
                    w = gb[r, pl.ds(q * 16, 16)]
                    ev = plsc.bitcast(w << 16, jnp.float32)
                    sb[r, pl.ds(q * 32, 16)] = ev * wv
                    if q < 4:
                        od = plsc.bitcast(w & HMASK, jnp.float32)
                        sb[r, pl.ds(q * 32 + 16, 16)] = od * wv
            return c
        lax.fori_loop(0, KB // 16, tbody, 0)

    def step(j, b2, i4):
        b2p = (b2 + 1) % BSLOT
        i4p = (i4 + 1) % ISLOT
        i4n = (i4 + 2) % ISLOT

        @pl.when(j >= 2)
        def _():
            scat_wait(b2, i4n)
        gather_wait(b2, i4)

        @pl.when(j + 1 < NBLK)
        def _():
            idx_wait(j + 1, i4p)
            gather_start(b2p, i4p)

        @pl.when(j + 2 < NBLK)
        def _():
            idx_fetch(j + 2, i4n)
        scale(b2, i4)
        scat_start(b2, i4)

    idx_fetch(0, 0)
    idx_fetch(1, 1)
    idx_wait(0, 0)
    gather_start(0, 0)

    UNROLL = 4

    def round4(r, carry):
        j0 = r * UNROLL
        for u in range(UNROLL):
            step(j0 + u, u % BSLOT, u % ISLOT)
        return carry
    lax.fori_loop(0, NBLK // UNROLL, round4, 0)
    for j in range(NBLK // UNROLL * UNROLL, NBLK):
        step(jnp.int32(j), j % BSLOT, j % ISLOT)
    scat_wait((NBLK - 2) % BSLOT, (NBLK - 2) % ISLOT)
    scat_wait((NBLK - 1) % BSLOT, (NBLK - 1) % ISLOT)

    plsc.subcore_barrier()
    pltpu.sync_copy(ps.at[pl.ds(sid * nrows, nrows)],
                    out_hbm.at[cid, pl.ds(sid * nrows, nrows)])


def kernel(feat_in, edge_row, edge_col, adj_values, W0, b0, W1, b1, attention, scale, offset):
    att2 = attention.reshape(1, D)
    b0_2 = b0.reshape(1, D)
    b1_2 = b1.reshape(1, D)
    sc2 = scale.reshape(1, D)
    off2 = offset.reshape(1, D)

    g = pl.pallas_call(
        _prep_body,
        grid=(GRID,),
        in_specs=[
            pl.BlockSpec((ROWB, D), lambda i: (i, 0)),
            pl.BlockSpec((D, D), lambda i: (0, 0)),
            pl.BlockSpec((1, D), lambda i: (0, 0)),
            pl.BlockSpec((1, D), lambda i: (0, 0)),
        ],
        out_specs=pl.BlockSpec((ROWB, DG), lambda i: (i, 0)),
        out_shape=jax.ShapeDtypeStruct((N, DG), jnp.float32),
    )(feat_in, W0, b0_2, att2)

    f = g[:, :D]
    s_col = g[:, D:D + 1]
    halves = [(f[:, 32 * q:32 * q + 16], f[:, 32 * q + 16:32 * q + 32])
              for q in range(D // 32)]
    halves.append((jnp.broadcast_to(s_col, (N, 16)),
                   jnp.broadcast_to(s_col, (N, 16))))
    words = []
    for ev, od in halves:
        e16 = lax.bitcast_convert_type(ev.astype(jnp.bfloat16),
                                       jnp.uint16).astype(jnp.uint32)
        o16 = lax.bitcast_convert_type(od.astype(jnp.bfloat16),
                                       jnp.uint16).astype(jnp.uint32)
        words.append(lax.bitcast_convert_type(e16 | (o16 << 16), jnp.int32))
    g32 = jnp.concatenate(words, axis=1)

    partials = _agg(edge_row, edge_col, adj_values, g32)

    out = pl.pallas_call(
        _final_body,
        grid=(GRID,),
        in_specs=[
            pl.BlockSpec((ROWB, D), lambda i: (i, 0)),
            pl.BlockSpec((NC, ROWB, DG), lambda i: (0, i, 0)),
            pl.BlockSpec((D, D), lambda i: (0, 0)),
            pl.BlockSpec((1, D), lambda i: (0, 0)),
            pl.BlockSpec((1, D), lambda i: (0, 0)),
            pl.BlockSpec((1, D), lambda i: (0, 0)),
        ],
        out_specs=pl.BlockSpec((ROWB, D), lambda i: (i, 0)),
        out_shape=jax.ShapeDtypeStruct((N, D), jnp.float32),
    )(feat_in, partials, W1, b1_2, sc2, off2)
    return out


# speedup vs baseline: 1.6757x; 1.6757x over previous
import functools

import jax
import jax.numpy as jnp
from jax import lax
from jax.experimental import pallas as pl
from jax.experimental.pallas import tpu as pltpu
from jax.experimental.pallas import tpu_sc as plsc

N = 10000
E = 320000
D = 128
DW = 16
DG = D + DW
NC = 2
NS = 16
NW = NC * NS
EPW = E // NW
KB = 80
NBLK = EPW // KB
ROWB = 400
GRID = N // ROWB
NP = 10240
ZROWS = 16


def _prep_body(x_ref, w0_ref, b0_ref, att_ref, g_ref):
    x = x_ref[...]
    fs = lax.dot_general(x, w0_ref[...], (((1,), (1,)), ((), ())),
                         preferred_element_type=jnp.float32) + b0_ref[...]
    el = jnp.sum(fs * att_ref[...], axis=1, keepdims=True)
    el = jnp.where(el >= 0.0, el, 0.2 * el)
    s = jnp.exp(el)
    g_ref[:, :D] = fs * s
    g_ref[:, D:DG] = jnp.broadcast_to(s, (ROWB, DW))


def _final_body(x_ref, p_ref, w1_ref, b1_ref, sc_ref, off_ref, o_ref):
    x = x_ref[...]
    fself = lax.dot_general(x, w1_ref[...], (((1,), (1,)), ((), ())),
                            preferred_element_type=jnp.float32) + b1_ref[...]
    p = p_ref[...]
    tot = p[0] + p[1]
    den = jnp.max(tot[:, D:DG], axis=1, keepdims=True)
    aggr = tot[:, :D] / jnp.maximum(den, 1e-10)
    feat = jnp.maximum(aggr + fself, 0.0)
    m = jnp.mean(feat, axis=1, keepdims=True)
    d = feat - m
    v = jnp.mean(d * d, axis=1, keepdims=True)
    o_ref[...] = d * lax.rsqrt(v + 1e-9) * sc_ref[...] + off_ref[...]


_sc_mesh = plsc.VectorSubcoreMesh(core_axis_name="c", subcore_axis_name="s")


NSLOT = 3
ISLOT = 4


@functools.partial(
    pl.kernel,
    mesh=_sc_mesh,
    out_type=jax.ShapeDtypeStruct((NC, NP, DG), jnp.float32),
    scratch_types=[
        pltpu.VMEM((ISLOT, KB), jnp.int32),
        pltpu.VMEM((ISLOT, KB), jnp.int32),
        pltpu.VMEM((ISLOT, KB), jnp.float32),
        pltpu.VMEM((KB, DG), jnp.float32),
        pltpu.VMEM((KB, DG), jnp.float32),
        pltpu.VMEM((KB, DG), jnp.float32),
        pltpu.VMEM((ZROWS, DG), jnp.float32),
        pltpu.VMEM_SHARED((NP, DG), jnp.float32),
        pltpu.SemaphoreType.DMA,
        pltpu.SemaphoreType.DMA,
        pltpu.SemaphoreType.DMA,
        pltpu.SemaphoreType.DMA,
        pltpu.SemaphoreType.DMA,
        pltpu.SemaphoreType.DMA,
        pltpu.SemaphoreType.DMA,
        pltpu.SemaphoreType.DMA,
        pltpu.SemaphoreType.DMA,
        pltpu.SemaphoreType.DMA,
    ],
    compiler_params=pltpu.CompilerParams(use_tc_tiling_on_sc=False),
)
def _agg(rows_hbm, cols_hbm, adj_hbm, g_hbm, out_hbm,
         rowb, colb, adjb, buf0, buf1, buf2, zbuf, ps,
         isem0, isem1, isem2, isem3, gsem0, gsem1, gsem2,
         ssem0, ssem1, ssem2):
    cid = lax.axis_index("c")
    sid = lax.axis_index("s")
    wid = sid * NC + cid
    bufs = (buf0, buf1, buf2)
    isems = (isem0, isem1, isem2, isem3)
    gsems = (gsem0, gsem1, gsem2)
    ssems = (ssem0, ssem1, ssem2)
    zero16 = jnp.zeros((16,), jnp.float32)

    def zrow(i, carry):
        for q in range(DG // 16):
            zbuf[i, pl.ds(q * 16, 16)] = zero16
        return carry
    lax.fori_loop(0, ZROWS, zrow, 0)
    nrows = NP // NS

    def zchunk(k, carry):
        pltpu.sync_copy(zbuf, ps.at[pl.ds(sid * nrows + k * ZROWS, ZROWS)])
        return carry
    lax.fori_loop(0, nrows // ZROWS, zchunk, 0)
    plsc.subcore_barrier()

    def _eoff(j):
        return pl.multiple_of(wid * EPW + j * KB, 8)

    def idx_fetch(j, s):
        off = _eoff(j)
        pltpu.async_copy(rows_hbm.at[pl.ds(off, KB)], rowb.at[s], isems[s])
        pltpu.async_copy(cols_hbm.at[pl.ds(off, KB)], colb.at[s], isems[s])
        pltpu.async_copy(adj_hbm.at[pl.ds(off, KB)], adjb.at[s], isems[s])

    def idx_wait(j, s):
        off = _eoff(j)
        pltpu.make_async_copy(rows_hbm.at[pl.ds(off, KB)], rowb.at[s],
                              isems[s]).wait()
        pltpu.make_async_copy(cols_hbm.at[pl.ds(off, KB)], colb.at[s],
                              isems[s]).wait()
        pltpu.make_async_copy(adj_hbm.at[pl.ds(off, KB)], adjb.at[s],
                              isems[s]).wait()

    def gather_start(s3, i4):
        pltpu.async_copy(g_hbm.at[colb.at[i4]], bufs[s3], gsems[s3])

    def gather_wait(s3, i4):
        pltpu.make_async_copy(g_hbm.at[colb.at[i4]], bufs[s3],
                              gsems[s3]).wait()

    def scat_start(s3, i4):
        pltpu.async_copy(bufs[s3], ps.at[rowb.at[i4]], ssems[s3], add=True)

    def scat_wait(s3, i4):
        pltpu.make_async_copy(bufs[s3], ps.at[rowb.at[i4]], ssems[s3]).wait()

    def scale(s3, i4):
        buf = bufs[s3]

        def tbody(t, c):
            off = pl.multiple_of(t * 16, 16)
            av = adjb[i4, pl.ds(off, 16)]
            for i in range(16):
                wv = jnp.full((16,), av[i])
                r = t * 16 + i
                for q in range(DG // 16):
                    buf[r, pl.ds(q * 16, 16)] = buf[r, pl.ds(q * 16, 16)] * wv
            return c
        lax.fori_loop(0, KB // 16, tbody, 0)

    def step(j, s3, i4):
        s3p = (s3 + 1) % NSLOT
        i4p = (i4 + 1) % ISLOT
        i4n = (i4 + 2) % ISLOT

        @pl.when(j >= 2)
        def _():
            scat_wait(s3p, i4n)
        gather_wait(s3, i4)

        @pl.when(j + 1 < NBLK)
        def _():
            idx_wait(j + 1, i4p)
            gather_start(s3p, i4p)

        @pl.when(j + 2 < NBLK)
        def _():
            idx_fetch(j + 2, i4n)
        scale(s3, i4)
        scat_start(s3, i4)

    idx_fetch(0, 0)
    idx_fetch(1, 1)
    idx_wait(0, 0)
    gather_start(0, 0)

    UNROLL = 12

    def round12(r, carry):
        j0 = r * UNROLL
        for u in range(UNROLL):
            step(j0 + u, u % NSLOT, u % ISLOT)
        return carry
    lax.fori_loop(0, NBLK // UNROLL, round12, 0)
    for j in range(NBLK // UNROLL * UNROLL, NBLK):
        step(jnp.int32(j), j % NSLOT, j % ISLOT)
    scat_wait((NBLK - 2) % NSLOT, (NBLK - 2) % ISLOT)
    scat_wait((NBLK - 1) % NSLOT, (NBLK - 1) % ISLOT)

    plsc.subcore_barrier()
    pltpu.sync_copy(ps.at[pl.ds(sid * nrows, nrows)],
                    out_hbm.at[cid, pl.ds(sid * nrows, nrows)])


def kernel(feat_in, edge_row, edge_col, adj_values, W0, b0, W1, b1, attention, scale, offset):
    att2 = attention.reshape(1, D)
    b0_2 = b0.reshape(1, D)
    b1_2 = b1.reshape(1, D)
    sc2 = scale.reshape(1, D)
    off2 = offset.reshape(1, D)

    g = pl.pallas_call(
        _prep_body,
        grid=(GRID,),
        in_specs=[
            pl.BlockSpec((ROWB, D), lambda i: (i, 0)),
            pl.BlockSpec((D, D), lambda i: (0, 0)),
            pl.BlockSpec((1, D), lambda i: (0, 0)),
            pl.BlockSpec((1, D), lambda i: (0, 0)),
        ],
        out_specs=pl.BlockSpec((ROWB, DG), lambda i: (i, 0)),
        out_shape=jax.ShapeDtypeStruct((N, DG), jnp.float32),
    )(feat_in, W0, b0_2, att2)

    partials = _agg(edge_row, edge_col, adj_values, g)

    out = pl.pallas_call(
        _final_body,
        grid=(GRID,),
        in_specs=[
            pl.BlockSpec((ROWB, D), lambda i: (i, 0)),
            pl.BlockSpec((NC, ROWB, DG), lambda i: (0, i, 0)),
            pl.BlockSpec((D, D), lambda i: (0, 0)),
            pl.BlockSpec((1, D), lambda i: (0, 0)),
            pl.BlockSpec((1, D), lambda i: (0, 0)),
            pl.BlockSpec((1, D), lambda i: (0, 0)),
        ],
        out_specs=pl.BlockSpec((ROWB, D), lambda i: (i, 0)),
        out_shape=jax.ShapeDtypeStruct((N, D), jnp.float32),
    )(feat_in, partials, W1, b1_2, sc2, off2)
    return out
